# baseline (speedup 1.0000x reference)
import functools

import jax
import jax.numpy as jnp
from jax import lax
from jax.experimental import pallas as pl
from jax.experimental.pallas import tpu as pltpu
from jax.experimental.pallas import tpu_sc as plsc


def kernel(x, table):
    seq = x.shape[1]
    emb = table.shape[1]

    info = plsc.get_sparse_core_info()
    nc, ns = info.num_cores, info.num_subcores
    nw = nc * ns

    h = seq // 2
    assert h % nw == 0
    rows_per = h // nw
    half = rows_per // 2

    mesh = plsc.VectorSubcoreMesh(core_axis_name="c", subcore_axis_name="s")

    @functools.partial(
        pl.kernel,
        mesh=mesh,
        out_type=jax.ShapeDtypeStruct((h, emb), jnp.float32),
        scratch_types=[
            pltpu.VMEM((2, half, emb), jnp.float32),
            pltpu.SemaphoreType.DMA,
            pltpu.SemaphoreType.DMA,
            pltpu.SemaphoreType.DMA,
        ],
    )
    def sc_copy(table_hbm, out_hbm, buf, si0, si1, so):
        wid = lax.axis_index("s") * nc + lax.axis_index("c")
        base = wid * rows_per
        r0 = pltpu.async_copy(table_hbm.at[pl.ds(base, half)], buf.at[0], si0)
        r1 = pltpu.async_copy(
            table_hbm.at[pl.ds(base + half, half)], buf.at[1], si1
        )
        r0.wait()
        w0 = pltpu.async_copy(buf.at[0], out_hbm.at[pl.ds(base, half)], so)
        r1.wait()
        w1 = pltpu.async_copy(
            buf.at[1], out_hbm.at[pl.ds(base + half, half)], so
        )
        w0.wait()
        w1.wait()

    sc_part = sc_copy(table)

    block = 512
    nblocks = (seq - h) // block

    def tc_body(t_ref, o_ref):
        o_ref[...] = t_ref[...]

    tc_part = pl.pallas_call(
        tc_body,
        grid=(nblocks,),
        in_specs=[pl.BlockSpec((block, emb), lambda i, ob=h // block: (i + ob, 0))],
        out_specs=pl.BlockSpec((block, emb), lambda i: (i, 0)),
        out_shape=jax.ShapeDtypeStruct((seq - h, emb), jnp.float32),
    )(table)

    return jnp.concatenate([sc_part, tc_part], axis=0)


# speedup vs baseline: 1.1274x; 1.1274x over previous
import functools

import jax
import jax.numpy as jnp
from jax import lax
from jax.experimental import pallas as pl
from jax.experimental.pallas import tpu as pltpu
from jax.experimental.pallas import tpu_sc as plsc


def kernel(x, table):
    seq = x.shape[1]
    emb = table.shape[1]

    info = plsc.get_sparse_core_info()
    nc = info.num_cores
    assert seq % nc == 0
    rows_per = seq // nc
    half = rows_per // 2

    mesh = plsc.ScalarSubcoreMesh(axis_name="c", num_cores=nc)

    @functools.partial(
        pl.kernel,
        mesh=mesh,
        out_type=jax.ShapeDtypeStruct((seq, emb), jnp.float32),
        scratch_types=[
            pltpu.VMEM_SHARED((2, half, emb), jnp.float32),
            pltpu.SemaphoreType.DMA,
            pltpu.SemaphoreType.DMA,
            pltpu.SemaphoreType.DMA,
        ],
    )
    def sc_copy(table_hbm, out_hbm, buf, si0, si1, so):
        cid = lax.axis_index("c")
        base = cid * rows_per
        r0 = pltpu.async_copy(table_hbm.at[pl.ds(base, half)], buf.at[0], si0)
        r1 = pltpu.async_copy(
            table_hbm.at[pl.ds(base + half, half)], buf.at[1], si1
        )
        r0.wait()
        w0 = pltpu.async_copy(buf.at[0], out_hbm.at[pl.ds(base, half)], so)
        r1.wait()
        w1 = pltpu.async_copy(
            buf.at[1], out_hbm.at[pl.ds(base + half, half)], so
        )
        w0.wait()
        w1.wait()

    return sc_copy(table)
